# Initial kernel scaffold; baseline (speedup 1.0000x reference)
import jax
import jax.numpy as jnp
from jax.experimental import pallas as pl


def kernel(x, edge_index, set_indices, W_l0, b_l0, W_l1, b_l1, W_l2, b_l2, W_merge, b_merge, W_ff1, b_ff1, W_ff2, b_ff2):
    raise NotImplementedError("write your pallas kernel here")


# speedup vs baseline: 1.7177x; 1.7177x over previous
import functools

import jax
import jax.numpy as jnp
from jax import lax
from jax.experimental import pallas as pl
from jax.experimental.pallas import tpu as pltpu
from jax.experimental.pallas import tpu_sc as plsc

N = 10000
NP = 10240
E = 160000
EP = 163840
CB = 128
FC = 128
ZROW = 10000
JUNK = 10200
NC, NS = 2, 16
NW = NC * NS
EPW = EP // NW
NCHUNK = EPW // CB
RPS = NP // NS


def _make_sc_scatter():
    mesh = plsc.VectorSubcoreMesh(core_axis_name="c", subcore_axis_name="s")

    @functools.partial(
        pl.kernel,
        mesh=mesh,
        out_type=jax.ShapeDtypeStruct((NC * NP, FC), jnp.float32),
        scratch_types=[
            pltpu.VMEM((CB,), jnp.int32),
            pltpu.VMEM((CB,), jnp.int32),
            pltpu.VMEM((CB, FC), jnp.float32),
            pltpu.VMEM_SHARED((NP, FC), jnp.float32),
            pltpu.SemaphoreType.DMA,
        ],
    )
    def sc_scatter(src_hbm, dst_hbm, tab_hbm, zero_hbm, out_hbm,
                   sidx, didx, rows, acc, sem):
        c = lax.axis_index("c")
        s = lax.axis_index("s")
        wid = s * NC + c
        pltpu.sync_copy(zero_hbm.at[pl.ds(s * RPS, RPS)],
                        acc.at[pl.ds(s * RPS, RPS)])
        plsc.subcore_barrier()
        base = wid * EPW

        def body(j, carry):
            off = base + j * CB
            pltpu.sync_copy(src_hbm.at[pl.ds(off, CB)], sidx)
            pltpu.sync_copy(dst_hbm.at[pl.ds(off, CB)], didx)
            pltpu.async_copy(tab_hbm.at[sidx], rows, sem).wait()
            pltpu.sync_copy(rows, acc.at[didx], add=True)
            return carry

        lax.fori_loop(0, NCHUNK, body, 0)
        plsc.subcore_barrier()
        pltpu.sync_copy(acc.at[pl.ds(s * RPS, RPS)],
                        out_hbm.at[pl.ds(c * NP + s * RPS, RPS)])

    return sc_scatter


_sc_scatter = _make_sc_scatter()


def _segment_sum(src_p, dst_p, table):
    f = table.shape[1]
    zeros = jnp.zeros((NP, FC), jnp.float32)
    p0s, p1s = [], []
    for c0 in range(0, f, FC):
        out = _sc_scatter(src_p, dst_p, table[:, c0:c0 + FC], zeros)
        p0s.append(out[:NP])
        p1s.append(out[NP:])
    if len(p0s) == 1:
        return p0s[0], p1s[0]
    return jnp.concatenate(p0s, axis=1), jnp.concatenate(p1s, axis=1)


def _dis_body(deg_ref, dis_ref):
    deg = deg_ref[...]
    dis_ref[...] = jnp.where(deg > 0.0, lax.rsqrt(deg), 0.0)


def _dis_from_deg(deg2d):
    return pl.pallas_call(
        _dis_body,
        out_shape=jax.ShapeDtypeStruct(deg2d.shape, jnp.float32),
    )(deg2d)


def _scale_body(x_ref, d_ref, o_ref):
    o_ref[...] = x_ref[...] * d_ref[...]


def _scale(x, dis_col):
    f = x.shape[1]
    return pl.pallas_call(
        _scale_body,
        grid=(NP // 256,),
        in_specs=[
            pl.BlockSpec((256, f), lambda i: (i, 0)),
            pl.BlockSpec((256, 1), lambda i: (i, 0)),
        ],
        out_specs=pl.BlockSpec((256, f), lambda i: (i, 0)),
        out_shape=jax.ShapeDtypeStruct((NP, f), jnp.float32),
    )(x, dis_col)


def _combine_body(p0_ref, p1_ref, d_ref, cur_ref, nxt_ref):
    d = d_ref[...]
    cur = d * (p0_ref[...] + p1_ref[...])
    cur_ref[...] = cur
    nxt_ref[...] = d * cur


def _combine(p0, p1, dis_col):
    f = p0.shape[1]
    return pl.pallas_call(
        _combine_body,
        grid=(NP // 256,),
        in_specs=[
            pl.BlockSpec((256, f), lambda i: (i, 0)),
            pl.BlockSpec((256, f), lambda i: (i, 0)),
            pl.BlockSpec((256, 1), lambda i: (i, 0)),
        ],
        out_specs=[
            pl.BlockSpec((256, f), lambda i: (i, 0)),
            pl.BlockSpec((256, f), lambda i: (i, 0)),
        ],
        out_shape=[
            jax.ShapeDtypeStruct((NP, f), jnp.float32),
            jax.ShapeDtypeStruct((NP, f), jnp.float32),
        ],
    )(p0, p1, dis_col)


def _layer_body(x_ref, w_ref, b_ref, d_ref, h_ref, hs_ref):
    acc = jnp.dot(x_ref[...], w_ref[...], preferred_element_type=jnp.float32)
    h = jnp.maximum(acc + b_ref[...], 0.0)
    h_ref[...] = h
    hs_ref[...] = h * d_ref[...]


def _layer_matmul(xcat, wcat, bsum, dis_col):
    kd = xcat.shape[1]
    return pl.pallas_call(
        _layer_body,
        grid=(NP // 256,),
        in_specs=[
            pl.BlockSpec((256, kd), lambda i: (i, 0)),
            pl.BlockSpec((kd, 512), lambda i: (0, 0)),
            pl.BlockSpec((1, 512), lambda i: (0, 0)),
            pl.BlockSpec((256, 1), lambda i: (i, 0)),
        ],
        out_specs=[
            pl.BlockSpec((256, 512), lambda i: (i, 0)),
            pl.BlockSpec((256, 512), lambda i: (i, 0)),
        ],
        out_shape=[
            jax.ShapeDtypeStruct((NP, 512), jnp.float32),
            jax.ShapeDtypeStruct((NP, 512), jnp.float32),
        ],
    )(xcat, wcat, bsum, dis_col)


def _readout_body(x0_ref, x1_ref, wd_ref, wm_ref, wx_ref, bm_ref,
                  w1_ref, b1_ref, w2_ref, b2_ref, o_ref):
    x0 = x0_ref[...]
    x1 = x1_ref[...]
    f32 = jnp.float32
    pooled = (
        jnp.dot(jnp.abs(x0 - x1), wd_ref[...], preferred_element_type=f32)
        + jnp.dot((x0 + x1) * 0.5, wm_ref[...], preferred_element_type=f32)
        + jnp.dot(jnp.maximum(x0, x1), wx_ref[...], preferred_element_type=f32)
        + bm_ref[...]
    )
    ff = jnp.maximum(
        jnp.dot(pooled, w1_ref[...], preferred_element_type=f32) + b1_ref[...],
        0.0)
    o_ref[...] = (
        jnp.dot(ff, w2_ref[...], preferred_element_type=f32) + b2_ref[...])


def _readout(x0, x1, wm, bm, w1, b1, w2, b2):
    return pl.pallas_call(
        _readout_body,
        out_shape=jax.ShapeDtypeStruct((128, 256), jnp.float32),
    )(x0, x1, wm[:512], wm[512:1024], wm[1024:], bm.reshape(1, 512),
      w1, b1.reshape(1, 512), w2, b2.reshape(1, 256))


def kernel(x, edge_index, set_indices, W_l0, b_l0, W_l1, b_l1, W_l2, b_l2,
           W_merge, b_merge, W_ff1, b_ff1, W_ff2, b_ff2):
    src = edge_index[0]
    dst = edge_index[1]
    npad = EP - E
    src_p = jnp.concatenate([src, jnp.full((npad,), ZROW, jnp.int32)])
    dst_p = jnp.concatenate([dst, jnp.full((npad,), JUNK, jnp.int32)])

    ones_tab = jnp.ones((NP, FC), jnp.float32)
    d0, d1 = _segment_sum(src_p, dst_p, ones_tab)
    deg = (d0[:, :1] + d1[:, :1]).reshape(NP // FC, FC)
    dis_col = _dis_from_deg(deg).reshape(NP, 1)

    h = jnp.pad(x, ((0, NP - N), (0, 0)))
    hs = _scale(h, dis_col)

    for W, b in ((W_l0, b_l0), (W_l1, b_l1), (W_l2, b_l2)):
        kp1, din, _ = W.shape
        curs = [h]
        t = hs
        for _k in range(kp1 - 1):
            p0, p1 = _segment_sum(src_p, dst_p, t)
            cur, t = _combine(p0, p1, dis_col)
            curs.append(cur)
        xcat = jnp.concatenate(curs, axis=1)
        wcat = W.reshape(kp1 * din, 512)
        bsum = jnp.sum(b, axis=0).reshape(1, 512)
        h, hs = _layer_matmul(xcat, wcat, bsum, dis_col)

    base = (jnp.arange(100, dtype=set_indices.dtype) * 100)[:, None]
    sib = base + set_indices
    xs = h[sib]
    x0 = jnp.pad(xs[:, 0, :], ((0, 28), (0, 0)))
    x1 = jnp.pad(xs[:, 1, :], ((0, 28), (0, 0)))
    out = _readout(x0, x1, W_merge, b_merge, W_ff1, b_ff1, W_ff2, b_ff2)
    return out[:100]


# speedup vs baseline: 1.9158x; 1.1153x over previous
import functools

import jax
import jax.numpy as jnp
from jax import lax
from jax.experimental import pallas as pl
from jax.experimental.pallas import tpu as pltpu
from jax.experimental.pallas import tpu_sc as plsc

N = 10000
NP = 10240
E = 160000
EP = 163840
CB = 128
FC = 128
ZROW = 10000
JUNK = 10200
NC, NS = 2, 16
NW = NC * NS
EPW = EP // NW
NCHUNK = EPW // CB
RPS = NP // NS


def _make_sc_scatter():
    mesh = plsc.VectorSubcoreMesh(core_axis_name="c", subcore_axis_name="s")

    @functools.partial(
        pl.kernel,
        mesh=mesh,
        out_type=jax.ShapeDtypeStruct((NC * NP, FC), jnp.float32),
        scratch_types=[
            pltpu.VMEM((CB,), jnp.int32),
            pltpu.VMEM((CB,), jnp.int32),
            pltpu.VMEM((CB,), jnp.int32),
            pltpu.VMEM((CB,), jnp.int32),
            pltpu.VMEM((CB, FC), jnp.float32),
            pltpu.VMEM((CB, FC), jnp.float32),
            pltpu.VMEM_SHARED((NP, FC), jnp.float32),
            pltpu.SemaphoreType.DMA,
            pltpu.SemaphoreType.DMA,
        ],
    )
    def sc_scatter(src_hbm, dst_hbm, tab_hbm, zero_hbm, out_hbm,
                   sidx0, didx0, sidx1, didx1, rows0, rows1, acc, sem0, sem1):
        c = lax.axis_index("c")
        s = lax.axis_index("s")
        wid = s * NC + c
        pltpu.sync_copy(zero_hbm.at[pl.ds(s * RPS, RPS)],
                        acc.at[pl.ds(s * RPS, RPS)])
        plsc.subcore_barrier()
        base = wid * EPW

        def body(j, carry):
            off0 = base + (2 * j) * CB
            off1 = off0 + CB
            pltpu.sync_copy(src_hbm.at[pl.ds(off0, CB)], sidx0)
            pltpu.sync_copy(dst_hbm.at[pl.ds(off0, CB)], didx0)
            cp0 = pltpu.async_copy(tab_hbm.at[sidx0], rows0, sem0)
            pltpu.sync_copy(src_hbm.at[pl.ds(off1, CB)], sidx1)
            pltpu.sync_copy(dst_hbm.at[pl.ds(off1, CB)], didx1)
            cp1 = pltpu.async_copy(tab_hbm.at[sidx1], rows1, sem1)
            cp0.wait()
            pltpu.sync_copy(rows0, acc.at[didx0], add=True)
            cp1.wait()
            pltpu.sync_copy(rows1, acc.at[didx1], add=True)
            return carry

        lax.fori_loop(0, NCHUNK // 2, body, 0)
        plsc.subcore_barrier()
        pltpu.sync_copy(acc.at[pl.ds(s * RPS, RPS)],
                        out_hbm.at[pl.ds(c * NP + s * RPS, RPS)])

    return sc_scatter


_sc_scatter = _make_sc_scatter()


def _segment_sum(src_p, dst_p, table):
    f = table.shape[1]
    zeros = jnp.zeros((NP, FC), jnp.float32)
    p0s, p1s = [], []
    for c0 in range(0, f, FC):
        out = _sc_scatter(src_p, dst_p, table[:, c0:c0 + FC], zeros)
        p0s.append(out[:NP])
        p1s.append(out[NP:])
    if len(p0s) == 1:
        return p0s[0], p1s[0]
    return jnp.concatenate(p0s, axis=1), jnp.concatenate(p1s, axis=1)


def _dis_body(deg_ref, dis_ref):
    deg = deg_ref[...]
    dis_ref[...] = jnp.where(deg > 0.0, lax.rsqrt(deg), 0.0)


def _dis_from_deg(deg2d):
    return pl.pallas_call(
        _dis_body,
        out_shape=jax.ShapeDtypeStruct(deg2d.shape, jnp.float32),
    )(deg2d)


def _scale_body(x_ref, d_ref, o_ref):
    o_ref[...] = x_ref[...] * d_ref[...]


def _scale(x, dis_col):
    f = x.shape[1]
    return pl.pallas_call(
        _scale_body,
        grid=(NP // 256,),
        in_specs=[
            pl.BlockSpec((256, f), lambda i: (i, 0)),
            pl.BlockSpec((256, 1), lambda i: (i, 0)),
        ],
        out_specs=pl.BlockSpec((256, f), lambda i: (i, 0)),
        out_shape=jax.ShapeDtypeStruct((NP, f), jnp.float32),
    )(x, dis_col)


def _combine_body(p0_ref, p1_ref, d_ref, cur_ref, nxt_ref):
    d = d_ref[...]
    cur = d * (p0_ref[...] + p1_ref[...])
    cur_ref[...] = cur
    nxt_ref[...] = d * cur


def _combine(p0, p1, dis_col):
    f = p0.shape[1]
    return pl.pallas_call(
        _combine_body,
        grid=(NP // 256,),
        in_specs=[
            pl.BlockSpec((256, f), lambda i: (i, 0)),
            pl.BlockSpec((256, f), lambda i: (i, 0)),
            pl.BlockSpec((256, 1), lambda i: (i, 0)),
        ],
        out_specs=[
            pl.BlockSpec((256, f), lambda i: (i, 0)),
            pl.BlockSpec((256, f), lambda i: (i, 0)),
        ],
        out_shape=[
            jax.ShapeDtypeStruct((NP, f), jnp.float32),
            jax.ShapeDtypeStruct((NP, f), jnp.float32),
        ],
    )(p0, p1, dis_col)


def _layer_body(x_ref, w_ref, b_ref, d_ref, h_ref, hs_ref):
    acc = jnp.dot(x_ref[...], w_ref[...], preferred_element_type=jnp.float32)
    h = jnp.maximum(acc + b_ref[...], 0.0)
    h_ref[...] = h
    hs_ref[...] = h * d_ref[...]


def _layer_matmul(xcat, wcat, bsum, dis_col):
    kd = xcat.shape[1]
    return pl.pallas_call(
        _layer_body,
        grid=(NP // 256,),
        in_specs=[
            pl.BlockSpec((256, kd), lambda i: (i, 0)),
            pl.BlockSpec((kd, 512), lambda i: (0, 0)),
            pl.BlockSpec((1, 512), lambda i: (0, 0)),
            pl.BlockSpec((256, 1), lambda i: (i, 0)),
        ],
        out_specs=[
            pl.BlockSpec((256, 512), lambda i: (i, 0)),
            pl.BlockSpec((256, 512), lambda i: (i, 0)),
        ],
        out_shape=[
            jax.ShapeDtypeStruct((NP, 512), jnp.float32),
            jax.ShapeDtypeStruct((NP, 512), jnp.float32),
        ],
    )(xcat, wcat, bsum, dis_col)


def _readout_body(x0_ref, x1_ref, wd_ref, wm_ref, wx_ref, bm_ref,
                  w1_ref, b1_ref, w2_ref, b2_ref, o_ref):
    x0 = x0_ref[...]
    x1 = x1_ref[...]
    f32 = jnp.float32
    pooled = (
        jnp.dot(jnp.abs(x0 - x1), wd_ref[...], preferred_element_type=f32)
        + jnp.dot((x0 + x1) * 0.5, wm_ref[...], preferred_element_type=f32)
        + jnp.dot(jnp.maximum(x0, x1), wx_ref[...], preferred_element_type=f32)
        + bm_ref[...]
    )
    ff = jnp.maximum(
        jnp.dot(pooled, w1_ref[...], preferred_element_type=f32) + b1_ref[...],
        0.0)
    o_ref[...] = (
        jnp.dot(ff, w2_ref[...], preferred_element_type=f32) + b2_ref[...])


def _readout(x0, x1, wm, bm, w1, b1, w2, b2):
    return pl.pallas_call(
        _readout_body,
        out_shape=jax.ShapeDtypeStruct((128, 256), jnp.float32),
    )(x0, x1, wm[:512], wm[512:1024], wm[1024:], bm.reshape(1, 512),
      w1, b1.reshape(1, 512), w2, b2.reshape(1, 256))


def kernel(x, edge_index, set_indices, W_l0, b_l0, W_l1, b_l1, W_l2, b_l2,
           W_merge, b_merge, W_ff1, b_ff1, W_ff2, b_ff2):
    src = edge_index[0]
    dst = edge_index[1]
    npad = EP - E
    src_p = jnp.concatenate([src, jnp.full((npad,), ZROW, jnp.int32)])
    dst_p = jnp.concatenate([dst, jnp.full((npad,), JUNK, jnp.int32)])

    ones_tab = jnp.ones((NP, FC), jnp.float32)
    d0, d1 = _segment_sum(src_p, dst_p, ones_tab)
    deg = (d0[:, :1] + d1[:, :1]).reshape(NP // FC, FC)
    dis_col = _dis_from_deg(deg).reshape(NP, 1)

    h = jnp.pad(x, ((0, NP - N), (0, 0)))
    hs = _scale(h, dis_col)

    for W, b in ((W_l0, b_l0), (W_l1, b_l1), (W_l2, b_l2)):
        kp1, din, _ = W.shape
        curs = [h]
        t = hs
        for _k in range(kp1 - 1):
            p0, p1 = _segment_sum(src_p, dst_p, t)
            cur, t = _combine(p0, p1, dis_col)
            curs.append(cur)
        xcat = jnp.concatenate(curs, axis=1)
        wcat = W.reshape(kp1 * din, 512)
        bsum = jnp.sum(b, axis=0).reshape(1, 512)
        h, hs = _layer_matmul(xcat, wcat, bsum, dis_col)

    base = (jnp.arange(100, dtype=set_indices.dtype) * 100)[:, None]
    sib = base + set_indices
    xs = h[sib]
    x0 = jnp.pad(xs[:, 0, :], ((0, 28), (0, 0)))
    x1 = jnp.pad(xs[:, 1, :], ((0, 28), (0, 0)))
    out = _readout(x0, x1, W_merge, b_merge, W_ff1, b_ff1, W_ff2, b_ff2)
    return out[:100]
